# baseline (speedup 1.0000x reference)
import functools

import jax
import jax.numpy as jnp
from jax import lax
from jax.experimental import pallas as pl
from jax.experimental.pallas import tpu as pltpu
from jax.experimental.pallas import tpu_sc as plsc

_VOCAB = 1000000
_EMBED = 64
_BATCH = 16384
_PRED = 20

_NC = 2
_NS = 16
_NW = _NC * _NS
_ROWS_W = _BATCH // _NW
_CHUNK = 32
_NCHUNK = _ROWS_W // _CHUNK
_DPAIR = _EMBED // 2

_LN2 = 0.6931471805599453


def _vlog(x):
    bits = lax.bitcast_convert_type(x, jnp.int32)
    e = ((bits >> 23) & 0xFF) - 127
    m = lax.bitcast_convert_type(
        (bits & 0x007FFFFF) | 0x3F800000, jnp.float32)
    big = m > 1.4142135381698608
    m = jnp.where(big, m * 0.5, m)
    ef = (e + big.astype(jnp.int32)).astype(jnp.float32)
    t = m - 1.0
    z = t / (t + 2.0)
    z2 = z * z
    s = 2.0 * z * (1.0 + z2 * (1.0 / 3.0 + z2 * (0.2 + z2 * (1.0 / 7.0))))
    return ef * _LN2 + s


def _unpack_pair(words):
    halves = plsc.bitcast(words, jnp.bfloat16)
    return plsc.unpack(halves, format=plsc.PackFormat.INTERLEAVED)


def _body(posu, posv, ut, vt, out, uidx, urows, vidx, vrows, accv, sem):
    c = lax.axis_index("c")
    s = lax.axis_index("s")
    wid = s * _NC + c
    lanes = lax.iota(jnp.int32, 16)

    def chunk_body(i, acc):
        row0 = wid * _ROWS_W + i * _CHUNK
        pltpu.sync_copy(posu.at[pl.ds(row0, _CHUNK)], uidx)
        pltpu.sync_copy(posv.at[pl.ds(0, _PRED), pl.ds(row0, _CHUNK)], vidx)

        cps = [pltpu.async_copy(ut.at[uidx], urows, sem)]
        for p in range(_PRED):
            cps.append(pltpu.async_copy(
                vt.at[vidx.at[p]], vrows.at[pl.ds(p * _CHUNK, _CHUNK)], sem))
        for cp in cps:
            cp.wait()

        for g in range(_CHUNK // 16):
            rowit = lanes + g * 16
            vb = [lanes + (p * _CHUNK + g * 16) for p in range(_PRED)]
            preds = []
            for half in range(2):
                ps = list(range(half * 10, half * 10 + 10))

                def t_body(t, pr):
                    tcol = jnp.zeros((16,), jnp.int32) + t
                    ue, uo = _unpack_pair(
                        plsc.load_gather(urows, [rowit, tcol]))
                    new = []
                    for j, p in enumerate(ps):
                        ve, vo = _unpack_pair(
                            plsc.load_gather(vrows, [vb[p], tcol]))
                        new.append(pr[j] + (ue * ve + uo * vo))
                    return tuple(new)

                pr = lax.fori_loop(
                    0, _DPAIR, t_body,
                    tuple(jnp.zeros((16,), jnp.float32) for _ in range(10)))
                preds.extend(pr)
            mx = preds[0]
            for p in range(1, _PRED):
                mx = jnp.maximum(mx, preds[p])
            ssum = jnp.exp(preds[0] - mx)
            for p in range(1, _PRED):
                ssum = ssum + jnp.exp(preds[p] - mx)
            acc = acc + (mx + _vlog(ssum) - preds[0])
        return acc

    acc = lax.fori_loop(0, _NCHUNK, chunk_body, jnp.zeros((16,), jnp.float32))
    accv[...] = acc
    pltpu.sync_copy(accv, out.at[wid])


@jax.jit
def kernel(pos_u, pos_neg_v, u_table, v_table):
    posu = pos_u.reshape(_BATCH)
    posv_t = pos_neg_v.T
    ut_pairs = lax.bitcast_convert_type(
        u_table.astype(jnp.bfloat16).reshape(_VOCAB, _DPAIR, 2), jnp.int32)
    vt_pairs = lax.bitcast_convert_type(
        v_table.astype(jnp.bfloat16).reshape(_VOCAB, _DPAIR, 2), jnp.int32)
    mesh = plsc.VectorSubcoreMesh(core_axis_name="c", subcore_axis_name="s")
    f = functools.partial(
        pl.kernel,
        out_type=jax.ShapeDtypeStruct((_NW, 16), jnp.float32),
        mesh=mesh,
        scratch_types=[
            pltpu.VMEM((_CHUNK,), jnp.int32),
            pltpu.VMEM((_CHUNK, _DPAIR), jnp.int32),
            pltpu.VMEM((_PRED, _CHUNK), jnp.int32),
            pltpu.VMEM((_CHUNK * _PRED, _DPAIR), jnp.int32),
            pltpu.VMEM((16,), jnp.float32),
            pltpu.SemaphoreType.DMA,
        ],
        compiler_params=pltpu.CompilerParams(
            needs_layout_passes=False, use_tc_tiling_on_sc=False),
    )(_body)
    partials = f(posu, posv_t, ut_pairs, vt_pairs)
    return jnp.sum(partials) / _BATCH


# speedup vs baseline: 2.3199x; 2.3199x over previous
import functools

import jax
import jax.numpy as jnp
from jax import lax
from jax.experimental import pallas as pl
from jax.experimental.pallas import tpu as pltpu
from jax.experimental.pallas import tpu_sc as plsc

_VOCAB = 1000000
_EMBED = 64
_BATCH = 16384
_PRED = 20

_NC = 2
_NS = 16
_NW = _NC * _NS
_ROWS_W = _BATCH // _NW
_CHUNK = 32
_NCHUNK = _ROWS_W // _CHUNK
_DPAIR = _EMBED // 2

_LN2 = 0.6931471805599453


def _vlog(x):
    bits = lax.bitcast_convert_type(x, jnp.int32)
    e = ((bits >> 23) & 0xFF) - 127
    m = lax.bitcast_convert_type(
        (bits & 0x007FFFFF) | 0x3F800000, jnp.float32)
    big = m > 1.4142135381698608
    m = jnp.where(big, m * 0.5, m)
    ef = (e + big.astype(jnp.int32)).astype(jnp.float32)
    t = m - 1.0
    z = t / (t + 2.0)
    z2 = z * z
    s = 2.0 * z * (1.0 + z2 * (1.0 / 3.0 + z2 * (0.2 + z2 * (1.0 / 7.0))))
    return ef * _LN2 + s


def _unpack_pair(words):
    halves = plsc.bitcast(words, jnp.bfloat16)
    return plsc.unpack(halves, format=plsc.PackFormat.INTERLEAVED)


def _body(posu, posv, ut, vt, out, uidx, urows, vidx, vrows, accv, sem):
    c = lax.axis_index("c")
    s = lax.axis_index("s")
    wid = s * _NC + c
    lanes = lax.iota(jnp.int32, 16)

    def chunk_body(i, acc):
        row0 = wid * _ROWS_W + i * _CHUNK
        pltpu.sync_copy(posu.at[pl.ds(row0, _CHUNK)], uidx)
        pltpu.sync_copy(posv.at[pl.ds(0, _PRED), pl.ds(row0, _CHUNK)], vidx)

        cps = [pltpu.async_copy(ut.at[uidx], urows, sem)]
        for p in range(_PRED):
            cps.append(pltpu.async_copy(
                vt.at[vidx.at[p]], vrows.at[pl.ds(p * _CHUNK, _CHUNK)], sem))
        for cp in cps:
            cp.wait()

        for g in range(_CHUNK // 16):
            rowit = lanes + g * 16
            vb = [lanes + (p * _CHUNK + g * 16) for p in range(_PRED)]
            preds = []
            for half in range(2):
                ps = list(range(half * 10, half * 10 + 10))

                def d_body(d, pr):
                    dcol = jnp.zeros((16,), jnp.int32) + d
                    uvec = plsc.load_gather(urows, [rowit, dcol])
                    return tuple(
                        pr[j] + uvec * plsc.load_gather(vrows, [vb[p], dcol])
                        for j, p in enumerate(ps))

                pr = lax.fori_loop(
                    0, _EMBED, d_body,
                    tuple(jnp.zeros((16,), jnp.float32) for _ in range(10)),
                    unroll=4)
                preds.extend(pr)
            mx = preds[0]
            for p in range(1, _PRED):
                mx = jnp.maximum(mx, preds[p])
            ssum = jnp.exp(preds[0] - mx)
            for p in range(1, _PRED):
                ssum = ssum + jnp.exp(preds[p] - mx)
            acc = acc + (mx + _vlog(ssum) - preds[0])
        return acc

    acc = lax.fori_loop(0, _NCHUNK, chunk_body, jnp.zeros((16,), jnp.float32))
    accv[...] = acc
    pltpu.sync_copy(accv, out.at[wid])


@jax.jit
def kernel(pos_u, pos_neg_v, u_table, v_table):
    posu = pos_u.reshape(_BATCH)
    posv_t = pos_neg_v.T
    mesh = plsc.VectorSubcoreMesh(core_axis_name="c", subcore_axis_name="s")
    f = functools.partial(
        pl.kernel,
        out_type=jax.ShapeDtypeStruct((_NW, 16), jnp.float32),
        mesh=mesh,
        scratch_types=[
            pltpu.VMEM((_CHUNK,), jnp.int32),
            pltpu.VMEM((_CHUNK, _EMBED), jnp.float32),
            pltpu.VMEM((_PRED, _CHUNK), jnp.int32),
            pltpu.VMEM((_CHUNK * _PRED, _EMBED), jnp.float32),
            pltpu.VMEM((16,), jnp.float32),
            pltpu.SemaphoreType.DMA,
        ],
        compiler_params=pltpu.CompilerParams(
            needs_layout_passes=False, use_tc_tiling_on_sc=False),
    )(_body)
    partials = f(posu, posv_t, u_table, v_table)
    return jnp.sum(partials) / _BATCH
